# baseline (speedup 1.0000x reference)
import functools

import jax
import jax.numpy as jnp
from jax import lax
from jax.experimental import pallas as pl
from jax.experimental.pallas import tpu as pltpu
from jax.experimental.pallas import tpu_sc as plsc

F = 26
TOK = 64
EMB = 64
HID = 128
B = 16384

NC = 2
NS = 16
NW = NC * NS
ROWS = F * B
RPW = ROWS // NW
CH = 128
NSTEP = RPW // CH


PW = 128


def _mlp_fold_body(t_ref, w1_ref, b1_ref, w2_ref, b2_ref, p_ref):
    t = t_ref[0]
    h = jnp.dot(t, w1_ref[0], preferred_element_type=jnp.float32) + b1_ref[0]
    h = jnp.maximum(h, 0.0)
    o = jnp.dot(h, w2_ref[0], preferred_element_type=jnp.float32) + b2_ref[0]
    p_ref[0] = jnp.concatenate(
        [o, jnp.zeros((TOK, PW - EMB), jnp.float32)], axis=1)


def _fold_tables(tables64, W1, b1, W2, b2):
    return pl.pallas_call(
        _mlp_fold_body,
        grid=(F,),
        in_specs=[
            pl.BlockSpec((1, TOK, EMB), lambda i: (i, 0, 0)),
            pl.BlockSpec((1, EMB, HID), lambda i: (i, 0, 0)),
            pl.BlockSpec((1, 1, HID), lambda i: (i, 0, 0)),
            pl.BlockSpec((1, HID, EMB), lambda i: (i, 0, 0)),
            pl.BlockSpec((1, 1, EMB), lambda i: (i, 0, 0)),
        ],
        out_specs=pl.BlockSpec((1, TOK, PW), lambda i: (i, 0, 0)),
        out_shape=jax.ShapeDtypeStruct((F, TOK, PW), jnp.float32),
    )(tables64, W1, b1.reshape(F, 1, HID), W2, b2.reshape(F, 1, EMB))


def _gidx_body(in_ref, g_ref):
    x = in_ref[...]
    xt = x.T
    safe = jnp.where((xt >= 0) & (xt < TOK), xt, 0)
    off = lax.broadcasted_iota(jnp.int32, (F, B), 0) * TOK
    g_ref[...] = safe + off


def _gather_indices(inputs):
    return pl.pallas_call(
        _gidx_body,
        out_shape=jax.ShapeDtypeStruct((F, B), jnp.int32),
    )(inputs)


NBUF = 2
LOOK = 1
GROUPS = NSTEP // NBUF


def _sc_gather_body(p_hbm, gidx_hbm, out_hbm, idx_v, *bufs_and_sems):
    rows_bufs = bufs_and_sems[:NBUF]
    ext_bufs = bufs_and_sems[NBUF:2 * NBUF]
    gsems = bufs_and_sems[2 * NBUF:3 * NBUF]
    ssems = bufs_and_sems[3 * NBUF:]
    wid = lax.axis_index("s") * NC + lax.axis_index("c")
    base = wid * RPW
    pltpu.sync_copy(gidx_hbm.at[pl.ds(wid * NSTEP, NSTEP)], idx_v)

    def issue_gather(s, bs):
        pltpu.async_copy(p_hbm.at[idx_v.at[s]], rows_bufs[bs], gsems[bs])

    def wait_gather(b):
        pltpu.make_async_copy(
            p_hbm.at[pl.ds(0, CH)], rows_bufs[b], gsems[b]).wait()

    def wait_scatter(b):
        pltpu.make_async_copy(
            ext_bufs[b], out_hbm.at[0, pl.ds(0, CH)], ssems[b]).wait()

    for s in range(LOOK):
        issue_gather(s, s % NBUF)

    @pl.loop(0, GROUPS)
    def _(g):
        for b in range(NBUF):
            j = g * NBUF + b
            s = j + LOOK
            bs = (b + LOOK) % NBUF

            @pl.when(s < NSTEP)
            def _():
                @pl.when(s >= NBUF)
                def _():
                    wait_scatter(bs)
                issue_gather(s, bs)

            wait_gather(b)
            rref = rows_bufs[b]
            eref = ext_bufs[b]

            @pl.loop(0, CH, unroll=8)
            def _(r):
                for c in range(EMB // 16):
                    eref[r, pl.ds(c * 16, 16)] = rref[r, pl.ds(c * 16, 16)]

            row0 = base + j * CH
            pltpu.async_copy(
                eref,
                out_hbm.at[row0 // B, pl.ds(row0 % B, CH)], ssems[b])

    for b in range(NBUF):
        wait_scatter(b)


def _sc_gather(p_flat, gidx2):
    mesh = plsc.VectorSubcoreMesh(core_axis_name="c", subcore_axis_name="s")
    run = functools.partial(
        pl.kernel,
        out_type=jax.ShapeDtypeStruct((F, B, EMB), jnp.float32),
        mesh=mesh,
        scratch_types=[
            pltpu.VMEM((NSTEP, CH), jnp.int32),
        ] + [pltpu.VMEM((CH, PW), jnp.float32)] * NBUF
          + [pltpu.VMEM((CH, EMB), jnp.float32)] * NBUF
          + [pltpu.SemaphoreType.DMA] * (2 * NBUF),
        compiler_params=pltpu.CompilerParams(use_tc_tiling_on_sc=True),
    )(_sc_gather_body)
    return run(p_flat, gidx2)


def kernel(inputs, tables, W1, b1, W2, b2):
    tables64 = lax.slice(tables, (0, 0, 0), (F, TOK, EMB))
    p = _fold_tables(tables64, W1, b1, W2, b2)
    gidx = _gather_indices(inputs)
    return _sc_gather(
        p.reshape(F * TOK, PW),
        gidx.reshape(ROWS // CH, CH),
    )


# speedup vs baseline: 1.2599x; 1.2599x over previous
import functools

import jax
import jax.numpy as jnp
from jax import lax
from jax.experimental import pallas as pl
from jax.experimental.pallas import tpu as pltpu
from jax.experimental.pallas import tpu_sc as plsc

F = 26
TOK = 64
EMB = 64
HID = 128
B = 16384

NC = 2
NS = 16
NW = NC * NS
ROWS = F * B
RPW = ROWS // NW
CH = 128
NSTEP = RPW // CH


PW = 128


def _mlp_fold_body(t_ref, w1_ref, b1_ref, w2_ref, b2_ref, p_ref):
    t = t_ref[0]
    h = jnp.dot(t, w1_ref[0], preferred_element_type=jnp.float32) + b1_ref[0]
    h = jnp.maximum(h, 0.0)
    o = jnp.dot(h, w2_ref[0], preferred_element_type=jnp.float32) + b2_ref[0]
    p_ref[0] = jnp.concatenate(
        [o, jnp.zeros((TOK, PW - EMB), jnp.float32)], axis=1)


def _fold_tables(tables64, W1, b1, W2, b2):
    return pl.pallas_call(
        _mlp_fold_body,
        grid=(F,),
        in_specs=[
            pl.BlockSpec((1, TOK, EMB), lambda i: (i, 0, 0)),
            pl.BlockSpec((1, EMB, HID), lambda i: (i, 0, 0)),
            pl.BlockSpec((1, 1, HID), lambda i: (i, 0, 0)),
            pl.BlockSpec((1, HID, EMB), lambda i: (i, 0, 0)),
            pl.BlockSpec((1, 1, EMB), lambda i: (i, 0, 0)),
        ],
        out_specs=pl.BlockSpec((1, TOK, PW), lambda i: (i, 0, 0)),
        out_shape=jax.ShapeDtypeStruct((F, TOK, PW), jnp.float32),
    )(tables64, W1, b1.reshape(F, 1, HID), W2, b2.reshape(F, 1, EMB))


def _gidx_body(in_ref, g_ref):
    x = in_ref[...]
    xt = x.T
    safe = jnp.where((xt >= 0) & (xt < TOK), xt, 0)
    off = lax.broadcasted_iota(jnp.int32, (F, B), 0) * TOK
    g_ref[...] = safe + off


def _gather_indices(inputs):
    return pl.pallas_call(
        _gidx_body,
        out_shape=jax.ShapeDtypeStruct((F, B), jnp.int32),
    )(inputs)


NG = 4
NE = 2
LOOK = 2
GROUPS = NSTEP // NG


def _sc_gather_body(p_hbm, gidx_hbm, out_hbm, idx_v, *bufs_and_sems):
    rows_bufs = bufs_and_sems[:NG]
    ext_bufs = bufs_and_sems[NG:NG + NE]
    gsems = bufs_and_sems[NG + NE:2 * NG + NE]
    ssems = bufs_and_sems[2 * NG + NE:]
    wid = lax.axis_index("s") * NC + lax.axis_index("c")
    base = wid * RPW
    pltpu.sync_copy(gidx_hbm.at[pl.ds(wid * NSTEP, NSTEP)], idx_v)

    def issue_gather(s, bs):
        pltpu.async_copy(p_hbm.at[idx_v.at[s]], rows_bufs[bs], gsems[bs])

    def wait_gather(b):
        pltpu.make_async_copy(
            p_hbm.at[pl.ds(0, CH)], rows_bufs[b], gsems[b]).wait()

    def wait_scatter(e):
        pltpu.make_async_copy(
            ext_bufs[e], out_hbm.at[pl.ds(0, CH)], ssems[e]).wait()

    for s in range(LOOK):
        issue_gather(s, s % NG)

    @pl.loop(0, GROUPS)
    def _(g):
        for b in range(NG):
            j = g * NG + b
            s = j + LOOK
            bs = (b + LOOK) % NG

            @pl.when(s < NSTEP)
            def _():
                issue_gather(s, bs)

            wait_gather(b)
            rref = rows_bufs[b]
            e = b % NE
            eref = ext_bufs[e]

            @pl.when(j >= NE)
            def _():
                wait_scatter(e)

            @pl.loop(0, CH, unroll=8)
            def _(r):
                for c in range(EMB // 16):
                    eref[r, pl.ds(c * 16, 16)] = rref[r, pl.ds(c * 16, 16)]

            pltpu.async_copy(
                eref, out_hbm.at[pl.ds(base + j * CH, CH)], ssems[e])

    for e in range(NE):
        wait_scatter(e)


def _sc_gather(p_flat, gidx2):
    mesh = plsc.VectorSubcoreMesh(core_axis_name="c", subcore_axis_name="s")
    run = functools.partial(
        pl.kernel,
        out_type=jax.ShapeDtypeStruct((ROWS, EMB), jnp.float32),
        mesh=mesh,
        scratch_types=[
            pltpu.VMEM((NSTEP, CH), jnp.int32),
        ] + [pltpu.VMEM((CH, PW), jnp.float32)] * NG
          + [pltpu.VMEM((CH, EMB), jnp.float32)] * NE
          + [pltpu.SemaphoreType.DMA] * (NG + NE),
        compiler_params=pltpu.CompilerParams(use_tc_tiling_on_sc=True),
    )(_sc_gather_body)
    return run(p_flat, gidx2)


def kernel(inputs, tables, W1, b1, W2, b2):
    tables64 = lax.slice(tables, (0, 0, 0), (F, TOK, EMB))
    p = _fold_tables(tables64, W1, b1, W2, b2)
    gidx = _gather_indices(inputs)
    out = _sc_gather(
        p.reshape(F * TOK, PW),
        gidx.reshape(ROWS // CH, CH),
    )
    return out.reshape(F, B, EMB)
